# baseline (speedup 1.0000x reference)
import functools

import jax
import jax.numpy as jnp
from jax import lax
from jax.experimental import pallas as pl
from jax.experimental.pallas import tpu as pltpu
from jax.experimental.pallas import tpu_sc as plsc

NBX = 512
NBY = 512
NBINS = NBX * NBY
BSX = 1.0 / NBX
BSY = 1.0 / NBY
INV_BSX = float(NBX)
INV_BSY = float(NBY)
BIN_AREA = BSX * BSY
H2 = BSX * BSY
N_ITERS = 40

NPAD = 1 << 20
NW = 32
PER_W = NPAD // NW
CHUNK = 2048
NCHUNK = PER_W // CHUNK
ROWS = CHUNK // 128
STRIPE = NBINS // 16


def _splat_body(px_hbm, py_hbm, sx_hbm, sy_hbm, out_hbm,
                pxb, pyb, sxb, syb,
                i00b, i10b, i01b, i11b,
                v00b, v10b, v01b, v11b,
                tmp, grid_sh, sem):
    cid = lax.axis_index("c")
    sid = lax.axis_index("s")
    wid = cid * 16 + sid

    @pl.loop(0, STRIPE // 16)
    def _(i):
        tmp[pl.ds(i * 16, 16)] = jnp.zeros((16,), jnp.float32)

    pltpu.sync_copy(tmp, grid_sh.at[pl.ds(sid * STRIPE, STRIPE)])
    plsc.subcore_barrier()

    @pl.loop(0, NCHUNK)
    def _(t):
        base = wid * PER_W + t * CHUNK
        pltpu.sync_copy(px_hbm.at[pl.ds(base, CHUNK)], pxb)
        pltpu.sync_copy(py_hbm.at[pl.ds(base, CHUNK)], pyb)
        pltpu.sync_copy(sx_hbm.at[pl.ds(base, CHUNK)], sxb)
        pltpu.sync_copy(sy_hbm.at[pl.ds(base, CHUNK)], syb)

        @pl.loop(0, ROWS)
        def _(r):
            for c in range(8):
                sl = pl.ds(r * 128 + c * 16, 16)
                dst = pl.ds(r * 128 + c * 16, 16)
                gx = pxb[sl] * INV_BSX - 0.5
                gy = pyb[sl] * INV_BSY - 0.5
                ix0 = gx.astype(jnp.int32)
                iy0 = gy.astype(jnp.int32)
                wx = jnp.clip(gx - ix0.astype(jnp.float32), 0.0, 1.0)
                wy = jnp.clip(gy - iy0.astype(jnp.float32), 0.0, 1.0)
                ix1 = jnp.minimum(ix0 + 1, NBX - 1)
                iy1 = jnp.minimum(iy0 + 1, NBY - 1)
                area = sxb[sl] * syb[sl]
                ax1 = wx * area
                ax0 = area - ax1
                bx0 = ix0 * NBY
                bx1 = ix1 * NBY
                i00b[dst] = bx0 + iy0
                i10b[dst] = bx1 + iy0
                i01b[dst] = bx0 + iy1
                i11b[dst] = bx1 + iy1
                v00b[dst] = ax0 - ax0 * wy
                v01b[dst] = ax0 * wy
                v10b[dst] = ax1 - ax1 * wy
                v11b[dst] = ax1 * wy
        c00 = pltpu.async_copy(v00b, grid_sh.at[i00b], sem, add=True)
        c10 = pltpu.async_copy(v10b, grid_sh.at[i10b], sem, add=True)
        c01 = pltpu.async_copy(v01b, grid_sh.at[i01b], sem, add=True)
        c11 = pltpu.async_copy(v11b, grid_sh.at[i11b], sem, add=True)
        c00.wait()
        c10.wait()
        c01.wait()
        c11.wait()

    plsc.subcore_barrier()

    pltpu.sync_copy(grid_sh.at[pl.ds(sid * STRIPE, STRIPE)], tmp)
    pltpu.sync_copy(tmp, out_hbm.at[pl.ds(cid * NBINS + sid * STRIPE, STRIPE)])


def _splat(px, py, sx, sy):
    mesh = plsc.VectorSubcoreMesh(core_axis_name="c", subcore_axis_name="s")
    k = pl.kernel(
        _splat_body,
        out_type=jax.ShapeDtypeStruct((2 * NBINS,), jnp.float32),
        mesh=mesh,
        scratch_types=[
            pltpu.VMEM((CHUNK,), jnp.float32),
            pltpu.VMEM((CHUNK,), jnp.float32),
            pltpu.VMEM((CHUNK,), jnp.float32),
            pltpu.VMEM((CHUNK,), jnp.float32),
            pltpu.VMEM((CHUNK,), jnp.int32),
            pltpu.VMEM((CHUNK,), jnp.int32),
            pltpu.VMEM((CHUNK,), jnp.int32),
            pltpu.VMEM((CHUNK,), jnp.int32),
            pltpu.VMEM((CHUNK,), jnp.float32),
            pltpu.VMEM((CHUNK,), jnp.float32),
            pltpu.VMEM((CHUNK,), jnp.float32),
            pltpu.VMEM((CHUNK,), jnp.float32),
            pltpu.VMEM((STRIPE,), jnp.float32),
            pltpu.VMEM_SHARED((NBINS,), jnp.float32),
            pltpu.SemaphoreType.DMA,
        ],
    )
    return k(px, py, sx, sy)


def _dense_body(p_ref, out_ref):
    rho = (p_ref[0] + p_ref[1]) * (1.0 / BIN_AREA)
    rhs = rho - jnp.mean(rho)

    def step(_, phi):
        up = jnp.concatenate([phi[:1, :], phi[:-1, :]], axis=0)
        down = jnp.concatenate([phi[1:, :], phi[-1:, :]], axis=0)
        left = jnp.concatenate([phi[:, :1], phi[:, :-1]], axis=1)
        right = jnp.concatenate([phi[:, 1:], phi[:, -1:]], axis=1)
        return 0.25 * (up + down + left + right - H2 * rhs)

    phi = lax.fori_loop(0, N_ITERS, step, jnp.zeros((NBX, NBY), jnp.float32))
    vx = jnp.concatenate([
        -(phi[1:2, :] - phi[0:1, :]) * INV_BSX,
        -(phi[2:, :] - phi[:-2, :]) * (0.5 * INV_BSX),
        -(phi[-1:, :] - phi[-2:-1, :]) * INV_BSX,
    ], axis=0)
    vy = jnp.concatenate([
        -(phi[:, 1:2] - phi[:, 0:1]) * INV_BSY,
        -(phi[:, 2:] - phi[:, :-2]) * (0.5 * INV_BSY),
        -(phi[:, -1:] - phi[:, -2:-1]) * INV_BSY,
    ], axis=1)
    energy = 0.5 * jnp.sum(rho * (vx * vx + vy * vy)) * BIN_AREA
    out_ref[...] = jnp.broadcast_to(energy, (1, 1))


def _dense(partials):
    return pl.pallas_call(
        _dense_body,
        out_shape=jax.ShapeDtypeStruct((1, 1), jnp.float32),
    )(partials)


def kernel(pos, node_size_x, node_size_y):
    n = pos.shape[0] // 2
    pad = NPAD - n
    px = jnp.concatenate([pos[:n], jnp.full((pad,), 0.5, jnp.float32)])
    py = jnp.concatenate([pos[n:], jnp.full((pad,), 0.5, jnp.float32)])
    sx = jnp.concatenate([node_size_x, jnp.zeros((pad,), jnp.float32)])
    sy = jnp.concatenate([node_size_y, jnp.zeros((pad,), jnp.float32)])
    flat = _splat(px, py, sx, sy)
    partials = flat.reshape(2, NBX, NBY)
    energy = _dense(partials)
    return energy.reshape(1)


# speedup vs baseline: 1.2157x; 1.2157x over previous
import functools

import jax
import jax.numpy as jnp
from jax import lax
from jax.experimental import pallas as pl
from jax.experimental.pallas import tpu as pltpu
from jax.experimental.pallas import tpu_sc as plsc

NBX = 512
NBY = 512
NBINS = NBX * NBY
BSX = 1.0 / NBX
BSY = 1.0 / NBY
INV_BSX = float(NBX)
INV_BSY = float(NBY)
BIN_AREA = BSX * BSY
H2 = BSX * BSY
N_ITERS = 40

NPAD = 1 << 20
NW = 32
PER_W = NPAD // NW
CHUNK = 2048
NCHUNK = PER_W // CHUNK
ROWS = CHUNK // 128
STRIPE = NBINS // 16


def _splat_body(px_hbm, py_hbm, sx_hbm, sy_hbm, out_hbm, *scratch):
    sets = [scratch[0:12], scratch[12:24]]
    tmp, grid_sh, lsem, ssem = scratch[24:28]
    cid = lax.axis_index("c")
    sid = lax.axis_index("s")
    wid = cid * 16 + sid

    @pl.loop(0, STRIPE // 16)
    def _(i):
        tmp[pl.ds(i * 16, 16)] = jnp.zeros((16,), jnp.float32)

    pltpu.sync_copy(tmp, grid_sh.at[pl.ds(sid * STRIPE, STRIPE)])
    plsc.subcore_barrier()

    def fire_loads(t, s):
        pxb, pyb, sxb, syb = s[0:4]
        base = wid * PER_W + t * CHUNK
        return [
            pltpu.async_copy(px_hbm.at[pl.ds(base, CHUNK)], pxb, lsem),
            pltpu.async_copy(py_hbm.at[pl.ds(base, CHUNK)], pyb, lsem),
            pltpu.async_copy(sx_hbm.at[pl.ds(base, CHUNK)], sxb, lsem),
            pltpu.async_copy(sy_hbm.at[pl.ds(base, CHUNK)], syb, lsem),
        ]

    def compute(s):
        pxb, pyb, sxb, syb, i00b, i10b, i01b, i11b, v00b, v10b, v01b, v11b = s

        @pl.loop(0, ROWS)
        def _(r):
            for c in range(8):
                sl = pl.ds(r * 128 + c * 16, 16)
                gx = pxb[sl] * INV_BSX - 0.5
                gy = pyb[sl] * INV_BSY - 0.5
                ix0 = gx.astype(jnp.int32)
                iy0 = gy.astype(jnp.int32)
                wx = jnp.clip(gx - ix0.astype(jnp.float32), 0.0, 1.0)
                wy = jnp.clip(gy - iy0.astype(jnp.float32), 0.0, 1.0)
                ix1 = jnp.minimum(ix0 + 1, NBX - 1)
                iy1 = jnp.minimum(iy0 + 1, NBY - 1)
                area = sxb[sl] * syb[sl]
                ax1 = wx * area
                ax0 = area - ax1
                bx0 = ix0 * NBY
                bx1 = ix1 * NBY
                i00b[sl] = bx0 + iy0
                i10b[sl] = bx1 + iy0
                i01b[sl] = bx0 + iy1
                i11b[sl] = bx1 + iy1
                v00b[sl] = ax0 - ax0 * wy
                v01b[sl] = ax0 * wy
                v10b[sl] = ax1 - ax1 * wy
                v11b[sl] = ax1 * wy

    def fire_scatters(s):
        i00b, i10b, i01b, i11b, v00b, v10b, v01b, v11b = s[4:12]
        return [
            pltpu.async_copy(v00b, grid_sh.at[i00b], ssem, add=True),
            pltpu.async_copy(v10b, grid_sh.at[i10b], ssem, add=True),
            pltpu.async_copy(v01b, grid_sh.at[i01b], ssem, add=True),
            pltpu.async_copy(v11b, grid_sh.at[i11b], ssem, add=True),
        ]

    loads = fire_loads(0, sets[0])
    pending = [None, None]
    for t in range(NCHUNK):
        p = t % 2
        for c in loads:
            c.wait()
        if t + 1 < NCHUNK:
            loads = fire_loads(t + 1, sets[1 - p])
        if pending[p] is not None:
            for c in pending[p]:
                c.wait()
        compute(sets[p])
        pending[p] = fire_scatters(sets[p])
    for p in range(2):
        for c in pending[p]:
            c.wait()

    plsc.subcore_barrier()

    pltpu.sync_copy(grid_sh.at[pl.ds(sid * STRIPE, STRIPE)], tmp)
    pltpu.sync_copy(tmp, out_hbm.at[pl.ds(cid * NBINS + sid * STRIPE, STRIPE)])


def _splat(px, py, sx, sy):
    mesh = plsc.VectorSubcoreMesh(core_axis_name="c", subcore_axis_name="s")
    k = pl.kernel(
        _splat_body,
        out_type=jax.ShapeDtypeStruct((2 * NBINS,), jnp.float32),
        mesh=mesh,
        scratch_types=(
            [pltpu.VMEM((CHUNK,), jnp.float32)] * 4
            + [pltpu.VMEM((CHUNK,), jnp.int32)] * 4
            + [pltpu.VMEM((CHUNK,), jnp.float32)] * 4
        ) * 2 + [
            pltpu.VMEM((STRIPE,), jnp.float32),
            pltpu.VMEM_SHARED((NBINS,), jnp.float32),
            pltpu.SemaphoreType.DMA,
            pltpu.SemaphoreType.DMA,
        ],
    )
    return k(px, py, sx, sy)


def _dense_body(p_ref, out_ref):
    rho = (p_ref[0] + p_ref[1]) * (1.0 / BIN_AREA)
    rhs = rho - jnp.mean(rho)

    def step(_, phi):
        up = jnp.concatenate([phi[:1, :], phi[:-1, :]], axis=0)
        down = jnp.concatenate([phi[1:, :], phi[-1:, :]], axis=0)
        left = jnp.concatenate([phi[:, :1], phi[:, :-1]], axis=1)
        right = jnp.concatenate([phi[:, 1:], phi[:, -1:]], axis=1)
        return 0.25 * (up + down + left + right - H2 * rhs)

    phi = lax.fori_loop(0, N_ITERS, step, jnp.zeros((NBX, NBY), jnp.float32))
    vx = jnp.concatenate([
        -(phi[1:2, :] - phi[0:1, :]) * INV_BSX,
        -(phi[2:, :] - phi[:-2, :]) * (0.5 * INV_BSX),
        -(phi[-1:, :] - phi[-2:-1, :]) * INV_BSX,
    ], axis=0)
    vy = jnp.concatenate([
        -(phi[:, 1:2] - phi[:, 0:1]) * INV_BSY,
        -(phi[:, 2:] - phi[:, :-2]) * (0.5 * INV_BSY),
        -(phi[:, -1:] - phi[:, -2:-1]) * INV_BSY,
    ], axis=1)
    energy = 0.5 * jnp.sum(rho * (vx * vx + vy * vy)) * BIN_AREA
    out_ref[...] = jnp.broadcast_to(energy, (1, 1))


def _dense(partials):
    return pl.pallas_call(
        _dense_body,
        out_shape=jax.ShapeDtypeStruct((1, 1), jnp.float32),
    )(partials)


def kernel(pos, node_size_x, node_size_y):
    n = pos.shape[0] // 2
    pad = NPAD - n
    px = jnp.concatenate([pos[:n], jnp.full((pad,), 0.5, jnp.float32)])
    py = jnp.concatenate([pos[n:], jnp.full((pad,), 0.5, jnp.float32)])
    sx = jnp.concatenate([node_size_x, jnp.zeros((pad,), jnp.float32)])
    sy = jnp.concatenate([node_size_y, jnp.zeros((pad,), jnp.float32)])
    flat = _splat(px, py, sx, sy)
    partials = flat.reshape(2, NBX, NBY)
    energy = _dense(partials)
    return energy.reshape(1)
